# baseline (device time: 12544 ns/iter reference)
import jax
import jax.numpy as jnp
from jax import lax
from jax.experimental import pallas as pl
from jax.experimental.pallas import tpu as pltpu

N_GLOBAL = 1024
EPS = 1e-5


def kernel(x, gamma, beta):
    m, n = x.shape

    def body(x_ref, gamma_ref, beta_ref, out_ref,
             stats_ref, recv_ref, send_sem, recv_sem):
        my_x = lax.axis_index("x")
        my_y = lax.axis_index("y")
        nbr = (my_x, 1 - my_y)

        barrier_sem = pltpu.get_barrier_semaphore()
        pl.semaphore_signal(
            barrier_sem, inc=1,
            device_id=nbr, device_id_type=pl.DeviceIdType.MESH,
        )
        pl.semaphore_wait(barrier_sem, 1)

        xv = x_ref[:, :]
        stats_ref[:, 0:1] = jnp.sum(xv, axis=1, keepdims=True)
        stats_ref[:, 1:2] = jnp.sum(xv * xv, axis=1, keepdims=True)

        rdma = pltpu.make_async_remote_copy(
            src_ref=stats_ref,
            dst_ref=recv_ref,
            send_sem=send_sem,
            recv_sem=recv_sem,
            device_id=nbr,
            device_id_type=pl.DeviceIdType.MESH,
        )
        rdma.start()
        rdma.wait()

        tot = stats_ref[:, :] + recv_ref[:, :]
        mean = tot[:, 0:1] / N_GLOBAL
        var = tot[:, 1:2] / N_GLOBAL - mean * mean
        inv = lax.rsqrt(var + EPS)
        out_ref[:, :] = (xv - mean) * inv * gamma_ref[:, :] + beta_ref[:, :]

    return pl.pallas_call(
        body,
        out_shape=jax.ShapeDtypeStruct((m, n), x.dtype),
        in_specs=[pl.BlockSpec(memory_space=pltpu.VMEM)] * 3,
        out_specs=pl.BlockSpec(memory_space=pltpu.VMEM),
        scratch_shapes=[
            pltpu.VMEM((m, 2), jnp.float32),
            pltpu.VMEM((m, 2), jnp.float32),
            pltpu.SemaphoreType.DMA,
            pltpu.SemaphoreType.DMA,
        ],
        compiler_params=pltpu.CompilerParams(collective_id=0),
    )(x, gamma.reshape(1, n), beta.reshape(1, n))


# device time: 12124 ns/iter; 1.0346x vs baseline; 1.0346x over previous
import jax
import jax.numpy as jnp
from jax import lax
from jax.experimental import pallas as pl
from jax.experimental.pallas import tpu as pltpu

N_GLOBAL = 1024
EPS = 1e-5
NCHUNK = 4


def kernel(x, gamma, beta):
    m, n = x.shape
    mc = m // NCHUNK

    def body(x_ref, gamma_ref, beta_ref, out_ref,
             stats_ref, recv_ref, send_sems, recv_sems):
        my_x = lax.axis_index("x")
        my_y = lax.axis_index("y")
        nbr = (my_x, 1 - my_y)

        barrier_sem = pltpu.get_barrier_semaphore()
        pl.semaphore_signal(
            barrier_sem, inc=1,
            device_id=nbr, device_id_type=pl.DeviceIdType.MESH,
        )

        def make_rdma(c):
            return pltpu.make_async_remote_copy(
                src_ref=stats_ref.at[c],
                dst_ref=recv_ref.at[c],
                send_sem=send_sems.at[c],
                recv_sem=recv_sems.at[c],
                device_id=nbr,
                device_id_type=pl.DeviceIdType.MESH,
            )

        def stats(c):
            xv = x_ref[pl.ds(c * mc, mc), :]
            stats_ref[c, :, 0:1] = jnp.sum(xv, axis=1, keepdims=True)
            stats_ref[c, :, 1:2] = jnp.sum(xv * xv, axis=1, keepdims=True)

        rdmas = [make_rdma(c) for c in range(NCHUNK)]

        stats(0)
        pl.semaphore_wait(barrier_sem, 1)
        rdmas[0].start()
        for c in range(1, NCHUNK):
            stats(c)
            rdmas[c].start()

        for c in range(NCHUNK):
            rdmas[c].wait_recv()
            xv = x_ref[pl.ds(c * mc, mc), :]
            tot = stats_ref[c, :, :] + recv_ref[c, :, :]
            mean = tot[:, 0:1] / N_GLOBAL
            var = tot[:, 1:2] / N_GLOBAL - mean * mean
            inv = lax.rsqrt(var + EPS)
            out_ref[pl.ds(c * mc, mc), :] = (
                (xv - mean) * inv * gamma_ref[:, :] + beta_ref[:, :]
            )

        for c in range(NCHUNK):
            rdmas[c].wait_send()

    return pl.pallas_call(
        body,
        out_shape=jax.ShapeDtypeStruct((m, n), x.dtype),
        in_specs=[pl.BlockSpec(memory_space=pltpu.VMEM)] * 3,
        out_specs=pl.BlockSpec(memory_space=pltpu.VMEM),
        scratch_shapes=[
            pltpu.VMEM((NCHUNK, mc, 2), jnp.float32),
            pltpu.VMEM((NCHUNK, mc, 2), jnp.float32),
            pltpu.SemaphoreType.DMA((NCHUNK,)),
            pltpu.SemaphoreType.DMA((NCHUNK,)),
        ],
        compiler_params=pltpu.CompilerParams(collective_id=0),
    )(x, gamma.reshape(1, n), beta.reshape(1, n))


# device time: 8598 ns/iter; 1.4589x vs baseline; 1.4101x over previous
import jax
import jax.numpy as jnp
from jax import lax
from jax.experimental import pallas as pl
from jax.experimental.pallas import tpu as pltpu

N_GLOBAL = 1024
EPS = 1e-5
NSPLIT = 2


def kernel(x, gamma, beta):
    m, n = x.shape
    mh = m // NSPLIT

    def body(x_ref, gamma_ref, beta_ref, out_ref,
             stats_ref, recv_ref, send_sems, recv_sems):
        my_x = lax.axis_index("x")
        my_y = lax.axis_index("y")
        nbr = (my_x, 1 - my_y)

        barrier_sem = pltpu.get_barrier_semaphore()
        pl.semaphore_signal(
            barrier_sem, inc=1,
            device_id=nbr, device_id_type=pl.DeviceIdType.MESH,
        )

        def stats(h):
            xv = x_ref[pl.ds(h * mh, mh), :]
            s = jnp.sum(xv, axis=1, keepdims=True)
            ss = jnp.sum(xv * xv, axis=1, keepdims=True)
            stats_ref[:, pl.ds(h * mh, mh)] = jnp.concatenate(
                [s, ss], axis=1).T

        def make_rdma(h):
            return pltpu.make_async_remote_copy(
                src_ref=stats_ref.at[:, pl.ds(h * mh, mh)],
                dst_ref=recv_ref.at[:, pl.ds(h * mh, mh)],
                send_sem=send_sems.at[h],
                recv_sem=recv_sems.at[h],
                device_id=nbr,
                device_id_type=pl.DeviceIdType.MESH,
            )

        rdmas = [make_rdma(h) for h in range(NSPLIT)]

        stats(0)
        pl.semaphore_wait(barrier_sem, 1)
        rdmas[0].start()
        for h in range(1, NSPLIT):
            stats(h)
            rdmas[h].start()

        for h in range(NSPLIT):
            lo = h * mh
            rdmas[h].wait_recv()
            tot = (stats_ref[:, pl.ds(lo, mh)]
                   + recv_ref[:, pl.ds(lo, mh)])
            mean_r = tot[0:1, :] / N_GLOBAL
            var_r = tot[1:2, :] / N_GLOBAL - mean_r * mean_r
            inv_r = lax.rsqrt(var_r + EPS)
            mi = jnp.concatenate([mean_r, inv_r], axis=0).T
            xv = x_ref[pl.ds(lo, mh), :]
            out_ref[pl.ds(lo, mh), :] = (
                (xv - mi[:, 0:1]) * mi[:, 1:2] * gamma_ref[:, :]
                + beta_ref[:, :]
            )

        for h in range(NSPLIT):
            rdmas[h].wait_send()

    return pl.pallas_call(
        body,
        out_shape=jax.ShapeDtypeStruct((m, n), x.dtype),
        in_specs=[pl.BlockSpec(memory_space=pltpu.VMEM)] * 3,
        out_specs=pl.BlockSpec(memory_space=pltpu.VMEM),
        scratch_shapes=[
            pltpu.VMEM((2, m), jnp.float32),
            pltpu.VMEM((2, m), jnp.float32),
            pltpu.SemaphoreType.DMA((NSPLIT,)),
            pltpu.SemaphoreType.DMA((NSPLIT,)),
        ],
        compiler_params=pltpu.CompilerParams(collective_id=0),
    )(x, gamma.reshape(1, n), beta.reshape(1, n))
